# baseline (speedup 1.0000x reference)
import functools

import jax
import jax.numpy as jnp
from jax import lax
from jax.experimental import pallas as pl
from jax.experimental.pallas import tpu as pltpu
from jax.experimental.pallas import tpu_sc as plsc

_C = 17
_H = 128
_W = 128
_BB = 8
_CB = _BB * _C
_G = _H // 8
_BIG = 1 << 30
_HW = _H * _W
_NEG = float("-inf")

_CHUNK = 64
_NC = _CHUNK * _C
_PER_W = 48
_PAD = 32 * _PER_W


def _tc_kernel(x_ref, s_ref, i_ref):
    run = jnp.full((_CB, 8, _W), _NEG, dtype=jnp.float32)
    gidx = jnp.zeros((_CB, 8, _W), dtype=jnp.int32)
    for g in range(_G):
        xg = x_ref[:, :, g * 8:(g + 1) * 8, :].reshape(_CB, 8, _W)
        gt = xg > run
        run = jnp.where(gt, xg, run)
        gidx = jnp.where(gt, g, gidx)

    s8 = lax.broadcasted_iota(jnp.int32, (1, 8, _W), 1)
    l8 = lax.broadcasted_iota(jnp.int32, (1, 8, _W), 2)
    flat = gidx * (8 * _W) + s8 * _W + l8

    m = jnp.max(jnp.max(run, axis=1), axis=1, keepdims=True)
    cand = jnp.where(run == m[:, :, None], flat, _BIG)
    idx = jnp.min(jnp.min(cand, axis=1), axis=1, keepdims=True)

    s_ref[...] = m.reshape(_BB, _C, 1)
    i_ref[...] = idx.reshape(_BB, _C, 1)


def _sc_kernel(idx_hbm, score_hbm, x1d_hbm, ox_hbm, oy_hbm,
               idx_v, score_v,
               rl_v, rr_v, ru_v, rd_v,
               gl_v, gr_v, gu_v, gd_v,
               ox_v, oy_v,
               sem_l, sem_r, sem_u, sem_d):
    wid = lax.axis_index("s") * 2 + lax.axis_index("c")
    base_pt = wid * _PER_W

    pltpu.sync_copy(idx_hbm.at[pl.ds(base_pt, _PER_W)], idx_v)
    pltpu.sync_copy(score_hbm.at[pl.ds(base_pt, _PER_W)], score_v)

    for c in range(_PER_W // 16):
        sl = pl.ds(c * 16, 16)
        iv = idx_v[sl]
        base_el = lax.shift_left(lax.shift_right_logical(iv, 14), 14)
        loc = jnp.bitwise_and(iv, _HW - 1)
        iy = lax.shift_right_logical(loc, 7)
        ix = jnp.bitwise_and(loc, _W - 1)

        rl_v[sl] = base_el + iy * _W + jnp.maximum(ix - 1, 0)
        rr_v[sl] = base_el + iy * _W + jnp.minimum(ix + 1, _W - 1)
        ru_v[sl] = base_el + jnp.maximum(iy - 1, 0) * _W + ix
        rd_v[sl] = base_el + jnp.minimum(iy + 1, _H - 1) * _W + ix

    cl = pltpu.async_copy(x1d_hbm.at[rl_v], gl_v, sem_l)
    cr = pltpu.async_copy(x1d_hbm.at[rr_v], gr_v, sem_r)
    cu = pltpu.async_copy(x1d_hbm.at[ru_v], gu_v, sem_u)
    cd = pltpu.async_copy(x1d_hbm.at[rd_v], gd_v, sem_d)
    cl.wait()
    cr.wait()
    cu.wait()
    cd.wait()

    for c in range(_PER_W // 16):
        sl = pl.ds(c * 16, 16)
        vl = gl_v[sl]
        vr = gr_v[sl]
        vu = gu_v[sl]
        vd = gd_v[sl]

        iv = idx_v[sl]
        loc = jnp.bitwise_and(iv, _HW - 1)
        iy = lax.shift_right_logical(loc, 7)
        ix = jnp.bitwise_and(loc, _W - 1)
        s = score_v[sl]
        pos = s > 0.0
        fx = jnp.where(pos, ix.astype(jnp.float32), 0.0)
        fy = jnp.where(pos, iy.astype(jnp.float32), 0.0)
        cond = pos & (ix > 0) & (ix < _W - 1) & (iy > 0) & (iy < _H - 1)
        dx = jnp.sign(vr - vl) * 0.25
        dy = jnp.sign(vd - vu) * 0.25
        ox_v[sl] = fx + jnp.where(cond, dx, 0.0)
        oy_v[sl] = fy + jnp.where(cond, dy, 0.0)

    pltpu.sync_copy(ox_v, ox_hbm.at[pl.ds(base_pt, _PER_W)])
    pltpu.sync_copy(oy_v, oy_hbm.at[pl.ds(base_pt, _PER_W)])


@functools.cache
def _get_sc_call():
    return pl.kernel(
        _sc_kernel,
        mesh=plsc.VectorSubcoreMesh(core_axis_name="c", subcore_axis_name="s"),
        out_type=[
            jax.ShapeDtypeStruct((_PAD,), jnp.float32),
            jax.ShapeDtypeStruct((_PAD,), jnp.float32),
        ],
        scratch_types=(
            [pltpu.VMEM((_PER_W,), jnp.int32),
             pltpu.VMEM((_PER_W,), jnp.float32)]
            + [pltpu.VMEM((_PER_W,), jnp.int32) for _ in range(4)]
            + [pltpu.VMEM((_PER_W,), jnp.float32) for _ in range(4)]
            + [pltpu.VMEM((_PER_W,), jnp.float32) for _ in range(2)]
            + [pltpu.SemaphoreType.DMA for _ in range(4)]
        ),
    )


def _tc_call(x, k):
    nblk = _CHUNK // _BB
    return pl.pallas_call(
        _tc_kernel,
        grid=(nblk,),
        in_specs=[pl.BlockSpec((_BB, _C, _H, _W),
                               lambda i, k=k: (k * nblk + i, 0, 0, 0))],
        out_specs=[
            pl.BlockSpec((_BB, _C, 1), lambda i: (i, 0, 0)),
            pl.BlockSpec((_BB, _C, 1), lambda i: (i, 0, 0)),
        ],
        out_shape=[
            jax.ShapeDtypeStruct((_CHUNK, _C, 1), jnp.float32),
            jax.ShapeDtypeStruct((_CHUNK, _C, 1), jnp.int32),
        ],
    )(x)


@jax.jit
def kernel(x):
    batch = x.shape[0]
    n_chunks = batch // _CHUNK
    x1d = x.reshape(-1)
    metas = [_tc_call(x, k) for k in range(n_chunks)]
    parts = []
    for k in range(n_chunks):
        score, idx = metas[k]
        score_f = score.reshape(_NC)
        idx_g = idx.reshape(_NC) + (jnp.arange(_NC, dtype=jnp.int32)
                                    + k * _NC) * _HW
        idx_p = jnp.pad(idx_g, (0, _PAD - _NC))
        score_p = jnp.pad(score_f, (0, _PAD - _NC))
        ox, oy = _get_sc_call()(idx_p, score_p, x1d)
        pts = jnp.stack(
            [ox[:_NC].reshape(_CHUNK, _C), oy[:_NC].reshape(_CHUNK, _C),
             score_f.reshape(_CHUNK, _C)], axis=2)
        parts.append(pts)
    return jnp.concatenate(parts, axis=0)


# speedup vs baseline: 1.3047x; 1.3047x over previous
import functools

import jax
import jax.numpy as jnp
from jax import lax
from jax.experimental import pallas as pl
from jax.experimental.pallas import tpu as pltpu
from jax.experimental.pallas import tpu_sc as plsc

_C = 17
_H = 128
_W = 128
_BB = 8
_CB = _BB * _C
_BIG = 1 << 30
_NPTS = 128 * _C
_PER_W = 80
_HW = _H * _W


def _tc_kernel(x_ref, s_ref, i_ref):
    xb = x_ref[...].reshape(_CB, _H, _W)
    cm = jnp.max(xb, axis=1)
    m = jnp.max(cm, axis=1, keepdims=True)
    r3 = lax.broadcasted_iota(jnp.int32, (1, _H, _W), 1)
    c3 = lax.broadcasted_iota(jnp.int32, (1, _H, _W), 2)
    flat = r3 * _W + c3
    cand = jnp.where(xb == m[:, :, None], flat, _BIG)
    idx = jnp.min(jnp.min(cand, axis=1), axis=1, keepdims=True)
    s_ref[...] = m.reshape(_BB, _C, 1)
    i_ref[...] = idx.reshape(_BB, _C, 1)


def _sc_kernel(idx_hbm, score_hbm, x1d_hbm, ox_hbm, oy_hbm,
               idx_v, score_v,
               rl_v, rr_v, ru_v, rd_v,
               gl_v, gr_v, gu_v, gd_v,
               ox_v, oy_v,
               sem_l, sem_r, sem_u, sem_d):
    wid = lax.axis_index("s") * 2 + lax.axis_index("c")
    base_pt = jnp.minimum(wid * _PER_W, _NPTS - _PER_W)

    pltpu.sync_copy(idx_hbm.at[pl.ds(base_pt, _PER_W)], idx_v)
    pltpu.sync_copy(score_hbm.at[pl.ds(base_pt, _PER_W)], score_v)

    iota16 = lax.iota(jnp.int32, 16)
    for c in range(_PER_W // 16):
        sl = pl.ds(c * 16, 16)
        iv = idx_v[sl]
        iy = lax.shift_right_logical(iv, 7)
        ix = jnp.bitwise_and(iv, _W - 1)
        pt = base_pt + c * 16 + iota16
        base_el = pt * _HW

        rl_v[sl] = base_el + iy * _W + jnp.maximum(ix - 1, 0)
        rr_v[sl] = base_el + iy * _W + jnp.minimum(ix + 1, _W - 1)
        ru_v[sl] = base_el + jnp.maximum(iy - 1, 0) * _W + ix
        rd_v[sl] = base_el + jnp.minimum(iy + 1, _H - 1) * _W + ix

    cl = pltpu.async_copy(x1d_hbm.at[rl_v], gl_v, sem_l)
    cr = pltpu.async_copy(x1d_hbm.at[rr_v], gr_v, sem_r)
    cu = pltpu.async_copy(x1d_hbm.at[ru_v], gu_v, sem_u)
    cd = pltpu.async_copy(x1d_hbm.at[rd_v], gd_v, sem_d)
    cl.wait()
    cr.wait()
    cu.wait()
    cd.wait()

    for c in range(_PER_W // 16):
        sl = pl.ds(c * 16, 16)
        vl = gl_v[sl]
        vr = gr_v[sl]
        vu = gu_v[sl]
        vd = gd_v[sl]

        iv = idx_v[sl]
        iy = lax.shift_right_logical(iv, 7)
        ix = jnp.bitwise_and(iv, _W - 1)
        s = score_v[sl]
        pos = s > 0.0
        fx = jnp.where(pos, ix.astype(jnp.float32), 0.0)
        fy = jnp.where(pos, iy.astype(jnp.float32), 0.0)
        cond = pos & (ix > 0) & (ix < _W - 1) & (iy > 0) & (iy < _H - 1)
        dx = jnp.sign(vr - vl) * 0.25
        dy = jnp.sign(vd - vu) * 0.25
        ox_v[sl] = fx + jnp.where(cond, dx, 0.0)
        oy_v[sl] = fy + jnp.where(cond, dy, 0.0)

    pltpu.sync_copy(ox_v, ox_hbm.at[pl.ds(base_pt, _PER_W)])
    pltpu.sync_copy(oy_v, oy_hbm.at[pl.ds(base_pt, _PER_W)])


@functools.cache
def _get_sc_call():
    return pl.kernel(
        _sc_kernel,
        mesh=plsc.VectorSubcoreMesh(core_axis_name="c", subcore_axis_name="s"),
        out_type=[
            jax.ShapeDtypeStruct((_NPTS,), jnp.float32),
            jax.ShapeDtypeStruct((_NPTS,), jnp.float32),
        ],
        scratch_types=(
            [pltpu.VMEM((_PER_W,), jnp.int32),
             pltpu.VMEM((_PER_W,), jnp.float32)]
            + [pltpu.VMEM((_PER_W,), jnp.int32) for _ in range(4)]
            + [pltpu.VMEM((_PER_W,), jnp.float32) for _ in range(4)]
            + [pltpu.VMEM((_PER_W,), jnp.float32) for _ in range(2)]
            + [pltpu.SemaphoreType.DMA for _ in range(4)]
        ),
    )


@jax.jit
def kernel(x):
    batch = x.shape[0]
    score, idx = pl.pallas_call(
        _tc_kernel,
        grid=(batch // _BB,),
        in_specs=[pl.BlockSpec((_BB, _C, _H, _W), lambda i: (i, 0, 0, 0))],
        out_specs=[
            pl.BlockSpec((_BB, _C, 1), lambda i: (i, 0, 0)),
            pl.BlockSpec((_BB, _C, 1), lambda i: (i, 0, 0)),
        ],
        out_shape=[
            jax.ShapeDtypeStruct((batch, _C, 1), jnp.float32),
            jax.ShapeDtypeStruct((batch, _C, 1), jnp.int32),
        ],
    )(x)

    n = batch * _C
    score_f = score.reshape(n)
    idx_f = idx.reshape(n)
    x1d = x.reshape(-1)

    ox, oy = _get_sc_call()(idx_f, score_f, x1d)
    pts = jnp.stack(
        [ox.reshape(batch, _C), oy.reshape(batch, _C),
         score_f.reshape(batch, _C)], axis=2)
    return pts
